# baseline (speedup 1.0000x reference)
import functools

import jax
import jax.numpy as jnp
from jax import lax
from jax.experimental import pallas as pl
from jax.experimental.pallas import tpu as pltpu
from jax.experimental.pallas import tpu_sc as plsc

N_NODES = 10000
N_EDGES = 320000
HIDDEN = 128
HEADS = 16
HEAD_DIM = HIDDEN // HEADS
SCALE = 1.0 / (HEAD_DIM ** 0.5)
PAYLOAD = HIDDEN + HEADS

NC = 2
NS = 16
NW = NC * NS
E_PER_W = N_EDGES // NW
CHUNK = 40
N_CHUNKS = E_PER_W // CHUNK
ACC_ROWS = 10240
ROWS_PER_TILE = ACC_ROWS // NS


def _qkv_body(x_ref, wq_ref, wkv_ref, bq_ref, bkv_ref, q_ref, kv_ref):
    xb = x_ref[...]
    q_ref[...] = jnp.dot(xb, wq_ref[...], preferred_element_type=jnp.float32) + bq_ref[...]
    kv_ref[...] = jnp.dot(xb, wkv_ref[...], preferred_element_type=jnp.float32) + bkv_ref[...]


def _qkv_project(x, wqt, wkvt, bq, bkv):
    blk = 1000
    grid = (N_NODES // blk,)
    return pl.pallas_call(
        _qkv_body,
        grid=grid,
        in_specs=[pl.BlockSpec((blk, HIDDEN), lambda i: (i, 0)),
                  pl.BlockSpec((HIDDEN, HIDDEN), lambda i: (0, 0)),
                  pl.BlockSpec((HIDDEN, 2 * HIDDEN), lambda i: (0, 0)),
                  pl.BlockSpec((1, HIDDEN), lambda i: (0, 0)),
                  pl.BlockSpec((1, 2 * HIDDEN), lambda i: (0, 0))],
        out_specs=[pl.BlockSpec((blk, HIDDEN), lambda i: (i, 0)),
                   pl.BlockSpec((blk, 2 * HIDDEN), lambda i: (i, 0))],
        out_shape=[jax.ShapeDtypeStruct((N_NODES, HIDDEN), jnp.float32),
                   jax.ShapeDtypeStruct((N_NODES, 2 * HIDDEN), jnp.float32)],
    )(x, wqt, wkvt, bq[None, :], bkv[None, :])


def _bias_body(e_ref, w_ref, b_ref, o_ref):
    o_ref[...] = jnp.dot(e_ref[...], w_ref[...], preferred_element_type=jnp.float32) + b_ref[...]


def _edge_bias(edge_emb, webt, beb):
    blk = 4000
    grid = (N_EDGES // blk,)
    return pl.pallas_call(
        _bias_body,
        grid=grid,
        in_specs=[pl.BlockSpec((blk, HIDDEN), lambda i: (i, 0)),
                  pl.BlockSpec((HIDDEN, HEADS), lambda i: (0, 0)),
                  pl.BlockSpec((1, HEADS), lambda i: (0, 0))],
        out_specs=pl.BlockSpec((blk, HEADS), lambda i: (i, 0)),
        out_shape=jax.ShapeDtypeStruct((N_EDGES, HEADS), jnp.float32),
    )(edge_emb, webt, beb[None, :])


def _final_body(p0_ref, p1_ref, wot_ref, bo_ref, o_ref):
    a = p0_ref[...] + p1_ref[...]
    bsum = a[:, :HIDDEN]
    ssum = a[:, HIDDEN:] + 1e-10
    den = jnp.concatenate([ssum] * HEAD_DIM, axis=1)
    o_ref[...] = jnp.dot(bsum / den, wot_ref[...],
                         preferred_element_type=jnp.float32) + bo_ref[...]


def _finalize(part, wot, bo):
    blk = 80
    nb = N_NODES // blk
    off = ACC_ROWS // blk
    grid = (nb,)
    return pl.pallas_call(
        _final_body,
        grid=grid,
        in_specs=[pl.BlockSpec((blk, PAYLOAD), lambda i: (i, 0)),
                  pl.BlockSpec((blk, PAYLOAD), lambda i: (i + off, 0)),
                  pl.BlockSpec((HIDDEN, HIDDEN), lambda i: (0, 0)),
                  pl.BlockSpec((1, HIDDEN), lambda i: (0, 0))],
        out_specs=pl.BlockSpec((blk, HIDDEN), lambda i: (i, 0)),
        out_shape=jax.ShapeDtypeStruct((N_NODES, HIDDEN), jnp.float32),
    )(part, part, wot, bo[None, :])


def _edge_pass_body(q_hbm, kv_hbm, bias_hbm, row_hbm, col_hbm,
                    part_hbm,
                    ar0, ar1, ac0, ac1, br0, br1,
                    q0, kv0, bb0, bb1, s0, s1,
                    acc,
                    sg0, ss0, ss1, si0, si1, sb0, sb1):
    ar = [ar0, ar1]
    ac = [ac0, ac1]
    br = [br0, br1]
    bb = [bb0, bb1]
    sbuf = [s0, s1]
    ss = [ss0, ss1]
    si = [si0, si1]
    sb = [sb0, sb1]

    cid = lax.axis_index("c")
    sid = lax.axis_index("s")
    wid = sid * NC + cid
    zeros16 = jnp.zeros((16,), jnp.float32)

    def _zs(i, c):
        s0[i // (PAYLOAD // 16), pl.ds((i % (PAYLOAD // 16)) * 16, 16)] = zeros16
        return c
    lax.fori_loop(0, CHUNK * (PAYLOAD // 16), _zs, 0)

    r0 = sid * ROWS_PER_TILE
    for j in range(ROWS_PER_TILE // CHUNK):
        pltpu.sync_copy(s0, acc.at[pl.ds(r0 + j * CHUNK, CHUNK)])
    plsc.subcore_barrier()

    ebase = wid * E_PER_W

    def _fetch_a(ci, b):
        off = ebase + ci * CHUNK
        pltpu.async_copy(row_hbm.at[pl.ds(off, CHUNK)], ar[b], si[b])
        pltpu.async_copy(col_hbm.at[pl.ds(off, CHUNK)], ac[b], si[b])
        pltpu.async_copy(bias_hbm.at[pl.ds(off, CHUNK)], bb[b], si[b])

    def _wait_a(b):
        pltpu.make_async_copy(row_hbm.at[pl.ds(0, CHUNK)], ar[b], si[b]).wait()
        pltpu.make_async_copy(col_hbm.at[pl.ds(0, CHUNK)], ac[b], si[b]).wait()
        pltpu.make_async_copy(bias_hbm.at[pl.ds(0, CHUNK)], bb[b], si[b]).wait()

    def _fetch_b(ci, b):
        off = ebase + ci * CHUNK
        pltpu.async_copy(row_hbm.at[pl.ds(off, CHUNK)], br[b], sb[b])

    def _wait_b(b):
        pltpu.make_async_copy(row_hbm.at[pl.ds(0, CHUNK)], br[b], sb[b]).wait()

    def _issue_gathers(b):
        pltpu.async_copy(q_hbm.at[ar[b]], q0, sg0)
        pltpu.async_copy(kv_hbm.at[ac[b]], kv0, sg0)

    def _wait_gathers(b):
        pltpu.make_async_copy(q_hbm.at[ar[b]], q0, sg0).wait()
        pltpu.make_async_copy(kv_hbm.at[ac[b]], kv0, sg0).wait()

    def _issue_scatter(b):
        pltpu.async_copy(sbuf[b], acc.at[br[b]], ss[b], add=True)

    def _wait_scatter(b):
        pltpu.make_async_copy(sbuf[b], acc.at[br[b]], ss[b]).wait()

    def _compute(b):
        bir, sr = bb[b], sbuf[b]

        @plsc.parallel_loop(0, CHUNK, unroll=2)
        def _edge(e):
            dot = q0[e, pl.ds(0, 16)] * kv0[e, pl.ds(0, 16)]
            for d in range(1, HEAD_DIM):
                dot = dot + q0[e, pl.ds(d * 16, 16)] * kv0[e, pl.ds(d * 16, 16)]
            p = jnp.exp(dot * SCALE + bir[e, :])
            for d in range(HEAD_DIM):
                sr[e, pl.ds(d * 16, 16)] = kv0[e, pl.ds(HIDDEN + d * 16, 16)] * p
            sr[e, pl.ds(HIDDEN, 16)] = p

    off0 = ebase
    pltpu.sync_copy(row_hbm.at[pl.ds(off0, CHUNK)], ar[0])
    pltpu.sync_copy(col_hbm.at[pl.ds(off0, CHUNK)], ac[0])
    pltpu.sync_copy(bias_hbm.at[pl.ds(off0, CHUNK)], bb[0])
    pltpu.sync_copy(row_hbm.at[pl.ds(off0, CHUNK)], br[0])
    _issue_gathers(0)
    _fetch_a(1, 1)

    def _pair(i, carry):
        for u in (0, 1):
            ci = 2 * i + u
            nu = 1 - u
            not_first = jnp.logical_or(i > 0, u > 0)
            _wait_gathers(u)
            _compute(u)

            @pl.when(not_first)
            def _():
                _wait_b(u)
            _issue_scatter(u)

            @pl.when(not_first)
            def _():
                _wait_scatter(nu)

            if u == 0:
                _wait_a(1)
                _issue_gathers(1)
                _fetch_b(ci + 1, 1)

                @pl.when(i < N_CHUNKS // 2 - 1)
                def _():
                    _fetch_a(ci + 2, 0)
            else:
                @pl.when(i < N_CHUNKS // 2 - 1)
                def _():
                    _wait_a(0)
                    _issue_gathers(0)
                    _fetch_b(ci + 1, 0)
                    _fetch_a(ci + 2, 1)
        return carry

    lax.fori_loop(0, N_CHUNKS // 2, _pair, 0)
    _wait_scatter(1)

    plsc.subcore_barrier()
    out0 = cid * ACC_ROWS + r0
    for j in range(ROWS_PER_TILE // CHUNK):
        pltpu.sync_copy(acc.at[pl.ds(r0 + j * CHUNK, CHUNK)], s0)
        pltpu.sync_copy(s0, part_hbm.at[pl.ds(out0 + j * CHUNK, CHUNK)])


def _edge_pass(qt, kvt, bias, row, col):
    mesh = plsc.VectorSubcoreMesh(core_axis_name="c", subcore_axis_name="s")
    fn = pl.kernel(
        _edge_pass_body,
        out_type=[jax.ShapeDtypeStruct((NC * ACC_ROWS, PAYLOAD), jnp.float32)],
        mesh=mesh,
        scratch_types=(
            [pltpu.VMEM((CHUNK,), jnp.int32)] * 6
            + [pltpu.VMEM((CHUNK, HIDDEN), jnp.float32)]
            + [pltpu.VMEM((CHUNK, 2 * HIDDEN), jnp.float32)]
            + [pltpu.VMEM((CHUNK, HEADS), jnp.float32)] * 2
            + [pltpu.VMEM((CHUNK, PAYLOAD), jnp.float32)] * 2
            + [pltpu.VMEM_SHARED((ACC_ROWS, PAYLOAD), jnp.float32)]
            + [pltpu.SemaphoreType.DMA] * 7
        ),
        compiler_params=pltpu.CompilerParams(use_tc_tiling_on_sc=False),
    )
    return fn(qt, kvt, bias, row, col)


def kernel(x, edge_index, edge_emb, Wq, bq, Wk, bk, Wv, bv, Wo, bo, Web, beb):
    c = jnp.arange(HIDDEN)
    perm = (c % HEADS) * HEAD_DIM + c // HEADS
    wqt = Wq.T[:, perm]
    wkvt = jnp.concatenate([Wk.T[:, perm], Wv.T[:, perm]], axis=1)
    wot = Wo.T[perm, :]
    bq_p = bq[perm]
    bkv_p = jnp.concatenate([bk[perm], bv[perm]])

    row = edge_index[0].astype(jnp.int32)
    col = edge_index[1].astype(jnp.int32)

    qt, kvt = _qkv_project(x, wqt, wkvt, bq_p, bkv_p)
    bias = _edge_bias(edge_emb, Web.T, beb)
    part, = _edge_pass(qt, kvt, bias, row, col)
    return _finalize(part, wot, bo)


# speedup vs baseline: 1.1925x; 1.1925x over previous
import functools

import jax
import jax.numpy as jnp
from jax import lax
from jax.experimental import pallas as pl
from jax.experimental.pallas import tpu as pltpu
from jax.experimental.pallas import tpu_sc as plsc

N_NODES = 10000
N_EDGES = 320000
HIDDEN = 128
HEADS = 16
HEAD_DIM = HIDDEN // HEADS
SCALE = 1.0 / (HEAD_DIM ** 0.5)
PAYLOAD = HIDDEN + HEADS

NC = 2
NS = 16
NW = NC * NS
E_PER_W = N_EDGES // NW
CHUNK = 40
N_CHUNKS = E_PER_W // CHUNK
ACC_ROWS = 10240
ROWS_PER_TILE = ACC_ROWS // NS


def _qkv_body(x_ref, wq_ref, wkv_ref, bq_ref, bkv_ref, q_ref, kv_ref):
    xb = x_ref[...]
    q_ref[...] = jnp.dot(xb, wq_ref[...], preferred_element_type=jnp.float32) + bq_ref[...]
    kv_ref[...] = jnp.dot(xb, wkv_ref[...], preferred_element_type=jnp.float32) + bkv_ref[...]


def _qkv_project(x, wqt, wkvt, bq, bkv):
    blk = 1000
    grid = (N_NODES // blk,)
    return pl.pallas_call(
        _qkv_body,
        grid=grid,
        in_specs=[pl.BlockSpec((blk, HIDDEN), lambda i: (i, 0)),
                  pl.BlockSpec((HIDDEN, HIDDEN), lambda i: (0, 0)),
                  pl.BlockSpec((HIDDEN, 2 * HIDDEN), lambda i: (0, 0)),
                  pl.BlockSpec((1, HIDDEN), lambda i: (0, 0)),
                  pl.BlockSpec((1, 2 * HIDDEN), lambda i: (0, 0))],
        out_specs=[pl.BlockSpec((blk, HIDDEN), lambda i: (i, 0)),
                   pl.BlockSpec((blk, 2 * HIDDEN), lambda i: (i, 0))],
        out_shape=[jax.ShapeDtypeStruct((N_NODES, HIDDEN), jnp.float32),
                   jax.ShapeDtypeStruct((N_NODES, 2 * HIDDEN), jnp.float32)],
    )(x, wqt, wkvt, bq[None, :], bkv[None, :])


def _bias_body(e_ref, w_ref, b_ref, o_ref):
    o_ref[...] = jnp.dot(e_ref[...], w_ref[...], preferred_element_type=jnp.float32) + b_ref[...]


def _edge_bias(edge_emb, webt, beb):
    blk = 4000
    grid = (N_EDGES // blk,)
    return pl.pallas_call(
        _bias_body,
        grid=grid,
        in_specs=[pl.BlockSpec((blk, HIDDEN), lambda i: (i, 0)),
                  pl.BlockSpec((HIDDEN, HEADS), lambda i: (0, 0)),
                  pl.BlockSpec((1, HEADS), lambda i: (0, 0))],
        out_specs=pl.BlockSpec((blk, HEADS), lambda i: (i, 0)),
        out_shape=jax.ShapeDtypeStruct((N_EDGES, HEADS), jnp.float32),
    )(edge_emb, webt, beb[None, :])


def _final_body(p0_ref, p1_ref, wot_ref, bo_ref, o_ref):
    a = p0_ref[...] + p1_ref[...]
    bsum = a[:, :HIDDEN]
    ssum = a[:, HIDDEN:] + 1e-10
    den = jnp.concatenate([ssum] * HEAD_DIM, axis=1)
    o_ref[...] = jnp.dot(bsum / den, wot_ref[...],
                         preferred_element_type=jnp.float32) + bo_ref[...]


def _finalize(part, wot, bo):
    blk = 80
    nb = N_NODES // blk
    off = ACC_ROWS // blk
    grid = (nb,)
    return pl.pallas_call(
        _final_body,
        grid=grid,
        in_specs=[pl.BlockSpec((blk, PAYLOAD), lambda i: (i, 0)),
                  pl.BlockSpec((blk, PAYLOAD), lambda i: (i + off, 0)),
                  pl.BlockSpec((HIDDEN, HIDDEN), lambda i: (0, 0)),
                  pl.BlockSpec((1, HIDDEN), lambda i: (0, 0))],
        out_specs=pl.BlockSpec((blk, HIDDEN), lambda i: (i, 0)),
        out_shape=jax.ShapeDtypeStruct((N_NODES, HIDDEN), jnp.float32),
    )(part, part, wot, bo[None, :])


def _edge_pass_body(q_hbm, kv_hbm, bias_hbm, row_hbm, col_hbm,
                    part_hbm,
                    ar0, ar1, ac0, ac1, br0, br1,
                    q0, q1, kv0, kv1, bb0, bb1, s0,
                    acc,
                    sg0, sg1, ss0, ss1, si0, si1, sb0, sb1):
    ar = [ar0, ar1]
    ac = [ac0, ac1]
    br = [br0, br1]
    bb = [bb0, bb1]
    qb = [q0, q1]
    kvb = [kv0, kv1]
    sg = [sg0, sg1]
    ss = [ss0, ss1]
    si = [si0, si1]
    sb = [sb0, sb1]

    cid = lax.axis_index("c")
    sid = lax.axis_index("s")
    wid = sid * NC + cid
    zeros16 = jnp.zeros((16,), jnp.float32)

    def _zs(i, c):
        s0[i // (PAYLOAD // 16), pl.ds((i % (PAYLOAD // 16)) * 16, 16)] = zeros16
        return c
    lax.fori_loop(0, CHUNK * (PAYLOAD // 16), _zs, 0)

    r0 = sid * ROWS_PER_TILE
    for j in range(ROWS_PER_TILE // CHUNK):
        pltpu.sync_copy(s0, acc.at[pl.ds(r0 + j * CHUNK, CHUNK)])
    plsc.subcore_barrier()

    ebase = wid * E_PER_W

    def _fetch_a(ci, b):
        off = ebase + ci * CHUNK
        pltpu.async_copy(row_hbm.at[pl.ds(off, CHUNK)], ar[b], si[b])
        pltpu.async_copy(col_hbm.at[pl.ds(off, CHUNK)], ac[b], si[b])
        pltpu.async_copy(bias_hbm.at[pl.ds(off, CHUNK)], bb[b], si[b])

    def _wait_a(b):
        pltpu.make_async_copy(row_hbm.at[pl.ds(0, CHUNK)], ar[b], si[b]).wait()
        pltpu.make_async_copy(col_hbm.at[pl.ds(0, CHUNK)], ac[b], si[b]).wait()
        pltpu.make_async_copy(bias_hbm.at[pl.ds(0, CHUNK)], bb[b], si[b]).wait()

    def _fetch_b(ci, b):
        off = ebase + ci * CHUNK
        pltpu.async_copy(row_hbm.at[pl.ds(off, CHUNK)], br[b], sb[b])

    def _wait_b(b):
        pltpu.make_async_copy(row_hbm.at[pl.ds(0, CHUNK)], br[b], sb[b]).wait()

    def _issue_gathers(b):
        pltpu.async_copy(q_hbm.at[ar[b]], qb[b], sg[b])
        pltpu.async_copy(kv_hbm.at[ac[b]], kvb[b], sg[b])

    def _wait_gathers(b):
        pltpu.make_async_copy(q_hbm.at[ar[b]], qb[b], sg[b]).wait()
        pltpu.make_async_copy(kv_hbm.at[ac[b]], kvb[b], sg[b]).wait()

    def _issue_scatter(b):
        pltpu.async_copy(s0, acc.at[br[b]], ss[b], add=True)

    def _wait_scatter(b):
        pltpu.make_async_copy(s0, acc.at[br[b]], ss[b]).wait()

    def _compute(b):
        bir, qr, kvr = bb[b], qb[b], kvb[b]

        @plsc.parallel_loop(0, CHUNK, unroll=2)
        def _edge(e):
            dot = qr[e, pl.ds(0, 16)] * kvr[e, pl.ds(0, 16)]
            for d in range(1, HEAD_DIM):
                dot = dot + qr[e, pl.ds(d * 16, 16)] * kvr[e, pl.ds(d * 16, 16)]
            p = jnp.exp(dot * SCALE + bir[e, :])
            for d in range(HEAD_DIM):
                s0[e, pl.ds(d * 16, 16)] = kvr[e, pl.ds(HIDDEN + d * 16, 16)] * p
            s0[e, pl.ds(HIDDEN, 16)] = p

    off0 = ebase
    pltpu.sync_copy(row_hbm.at[pl.ds(off0, CHUNK)], ar[0])
    pltpu.sync_copy(col_hbm.at[pl.ds(off0, CHUNK)], ac[0])
    pltpu.sync_copy(bias_hbm.at[pl.ds(off0, CHUNK)], bb[0])
    pltpu.sync_copy(row_hbm.at[pl.ds(off0, CHUNK)], br[0])
    _issue_gathers(0)
    _fetch_a(1, 1)

    def _pair(i, carry):
        for u in (0, 1):
            ci = 2 * i + u
            nu = 1 - u
            not_first = jnp.logical_or(i > 0, u > 0)
            not_last_pair = i < N_CHUNKS // 2 - 1
            _wait_gathers(u)

            if u == 0:
                _wait_a(1)
                _issue_gathers(1)
            else:
                @pl.when(not_last_pair)
                def _():
                    _wait_a(0)
                    _issue_gathers(0)

            @pl.when(not_first)
            def _():
                _wait_scatter(nu)
            if u == 0:
                _fetch_b(ci + 1, 1)
            else:
                @pl.when(not_last_pair)
                def _():
                    _fetch_b(ci + 1, 0)

            _compute(u)

            @pl.when(not_first)
            def _():
                _wait_b(u)
            _issue_scatter(u)

            @pl.when(not_last_pair)
            def _():
                _fetch_a(ci + 2, u)
        return carry

    lax.fori_loop(0, N_CHUNKS // 2, _pair, 0)
    _wait_scatter(1)

    plsc.subcore_barrier()
    out0 = cid * ACC_ROWS + r0
    for j in range(ROWS_PER_TILE // CHUNK):
        pltpu.sync_copy(acc.at[pl.ds(r0 + j * CHUNK, CHUNK)], s0)
        pltpu.sync_copy(s0, part_hbm.at[pl.ds(out0 + j * CHUNK, CHUNK)])


def _edge_pass(qt, kvt, bias, row, col):
    mesh = plsc.VectorSubcoreMesh(core_axis_name="c", subcore_axis_name="s")
    fn = pl.kernel(
        _edge_pass_body,
        out_type=[jax.ShapeDtypeStruct((NC * ACC_ROWS, PAYLOAD), jnp.float32)],
        mesh=mesh,
        scratch_types=(
            [pltpu.VMEM((CHUNK,), jnp.int32)] * 6
            + [pltpu.VMEM((CHUNK, HIDDEN), jnp.float32)] * 2
            + [pltpu.VMEM((CHUNK, 2 * HIDDEN), jnp.float32)] * 2
            + [pltpu.VMEM((CHUNK, HEADS), jnp.float32)] * 2
            + [pltpu.VMEM((CHUNK, PAYLOAD), jnp.float32)]
            + [pltpu.VMEM_SHARED((ACC_ROWS, PAYLOAD), jnp.float32)]
            + [pltpu.SemaphoreType.DMA] * 8
        ),
        compiler_params=pltpu.CompilerParams(use_tc_tiling_on_sc=False),
    )
    return fn(qt, kvt, bias, row, col)


def kernel(x, edge_index, edge_emb, Wq, bq, Wk, bk, Wv, bv, Wo, bo, Web, beb):
    c = jnp.arange(HIDDEN)
    perm = (c % HEADS) * HEAD_DIM + c // HEADS
    wqt = Wq.T[:, perm]
    wkvt = jnp.concatenate([Wk.T[:, perm], Wv.T[:, perm]], axis=1)
    wot = Wo.T[perm, :]
    bq_p = bq[perm]
    bkv_p = jnp.concatenate([bk[perm], bv[perm]])

    row = edge_index[0].astype(jnp.int32)
    col = edge_index[1].astype(jnp.int32)

    qt, kvt = _qkv_project(x, wqt, wkvt, bq_p, bkv_p)
    bias = _edge_bias(edge_emb, Web.T, beb)
    part, = _edge_pass(qt, kvt, bias, row, col)
    return _finalize(part, wot, bo)


# speedup vs baseline: 1.1934x; 1.0007x over previous
import functools

import jax
import jax.numpy as jnp
from jax import lax
from jax.experimental import pallas as pl
from jax.experimental.pallas import tpu as pltpu
from jax.experimental.pallas import tpu_sc as plsc

N_NODES = 10000
N_EDGES = 320000
HIDDEN = 128
HEADS = 16
HEAD_DIM = HIDDEN // HEADS
SCALE = 1.0 / (HEAD_DIM ** 0.5)
PAYLOAD = HIDDEN + HEADS

NC = 2
NS = 16
NW = NC * NS
E_PER_W = N_EDGES // NW
CHUNK = 40
N_CHUNKS = E_PER_W // CHUNK
ACC_ROWS = 10240
ROWS_PER_TILE = ACC_ROWS // NS


def _qkv_body(x_ref, wq_ref, wkv_ref, bq_ref, bkv_ref, q_ref, kv_ref):
    xb = x_ref[...]
    q_ref[...] = jnp.dot(xb, wq_ref[...], preferred_element_type=jnp.float32) + bq_ref[...]
    kv_ref[...] = jnp.dot(xb, wkv_ref[...], preferred_element_type=jnp.float32) + bkv_ref[...]


def _qkv_project(x, wqt, wkvt, bq, bkv):
    blk = 1000
    grid = (N_NODES // blk,)
    return pl.pallas_call(
        _qkv_body,
        grid=grid,
        in_specs=[pl.BlockSpec((blk, HIDDEN), lambda i: (i, 0)),
                  pl.BlockSpec((HIDDEN, HIDDEN), lambda i: (0, 0)),
                  pl.BlockSpec((HIDDEN, 2 * HIDDEN), lambda i: (0, 0)),
                  pl.BlockSpec((1, HIDDEN), lambda i: (0, 0)),
                  pl.BlockSpec((1, 2 * HIDDEN), lambda i: (0, 0))],
        out_specs=[pl.BlockSpec((blk, HIDDEN), lambda i: (i, 0)),
                   pl.BlockSpec((blk, 2 * HIDDEN), lambda i: (i, 0))],
        out_shape=[jax.ShapeDtypeStruct((N_NODES, HIDDEN), jnp.float32),
                   jax.ShapeDtypeStruct((N_NODES, 2 * HIDDEN), jnp.float32)],
    )(x, wqt, wkvt, bq[None, :], bkv[None, :])


def _bias_body(e_ref, w_ref, b_ref, o_ref):
    o_ref[...] = jnp.dot(e_ref[...], w_ref[...], preferred_element_type=jnp.float32) + b_ref[...]


def _edge_bias(edge_emb, webt, beb):
    blk = 4000
    grid = (N_EDGES // blk,)
    return pl.pallas_call(
        _bias_body,
        grid=grid,
        in_specs=[pl.BlockSpec((blk, HIDDEN), lambda i: (i, 0)),
                  pl.BlockSpec((HIDDEN, HEADS), lambda i: (0, 0)),
                  pl.BlockSpec((1, HEADS), lambda i: (0, 0))],
        out_specs=pl.BlockSpec((blk, HEADS), lambda i: (i, 0)),
        out_shape=jax.ShapeDtypeStruct((N_EDGES, HEADS), jnp.float32),
    )(edge_emb, webt, beb[None, :])


def _final_body(p0_ref, p1_ref, wot_ref, bo_ref, o_ref):
    a = p0_ref[...] + p1_ref[...]
    bsum = a[:, :HIDDEN]
    ssum = a[:, HIDDEN:] + 1e-10
    den = jnp.concatenate([ssum] * HEAD_DIM, axis=1)
    o_ref[...] = jnp.dot(bsum / den, wot_ref[...],
                         preferred_element_type=jnp.float32) + bo_ref[...]


def _finalize(part, wot, bo):
    blk = 80
    nb = N_NODES // blk
    off = ACC_ROWS // blk
    grid = (nb,)
    return pl.pallas_call(
        _final_body,
        grid=grid,
        in_specs=[pl.BlockSpec((blk, PAYLOAD), lambda i: (i, 0)),
                  pl.BlockSpec((blk, PAYLOAD), lambda i: (i + off, 0)),
                  pl.BlockSpec((HIDDEN, HIDDEN), lambda i: (0, 0)),
                  pl.BlockSpec((1, HIDDEN), lambda i: (0, 0))],
        out_specs=pl.BlockSpec((blk, HIDDEN), lambda i: (i, 0)),
        out_shape=jax.ShapeDtypeStruct((N_NODES, HIDDEN), jnp.float32),
    )(part, part, wot, bo[None, :])


def _edge_pass_body(q_hbm, kv_hbm, bias_hbm, row_hbm, col_hbm,
                    part_hbm,
                    ar0, ar1, ac0, ac1, br0, br1,
                    q0, q1, kv0, kv1, bb0, bb1, s0,
                    acc,
                    sg0, sg1, ss0, ss1, si0, si1, sb0, sb1):
    ar = [ar0, ar1]
    ac = [ac0, ac1]
    br = [br0, br1]
    bb = [bb0, bb1]
    qb = [q0, q1]
    kvb = [kv0, kv1]
    sg = [sg0, sg1]
    ss = [ss0, ss1]
    si = [si0, si1]
    sb = [sb0, sb1]

    cid = lax.axis_index("c")
    sid = lax.axis_index("s")
    wid = sid * NC + cid
    zeros16 = jnp.zeros((16,), jnp.float32)

    def _zs(i, c):
        s0[i // (PAYLOAD // 16), pl.ds((i % (PAYLOAD // 16)) * 16, 16)] = zeros16
        return c
    lax.fori_loop(0, CHUNK * (PAYLOAD // 16), _zs, 0)

    r0 = sid * ROWS_PER_TILE
    for j in range(ROWS_PER_TILE // CHUNK):
        pltpu.sync_copy(s0, acc.at[pl.ds(r0 + j * CHUNK, CHUNK)])
    plsc.subcore_barrier()

    ebase = wid * E_PER_W

    def _fetch_a(ci, b):
        off = ebase + ci * CHUNK
        pltpu.async_copy(row_hbm.at[pl.ds(off, CHUNK)], ar[b], si[b])
        pltpu.async_copy(col_hbm.at[pl.ds(off, CHUNK)], ac[b], si[b])
        pltpu.async_copy(bias_hbm.at[pl.ds(off, CHUNK)], bb[b], si[b])

    def _wait_a(b):
        pltpu.make_async_copy(row_hbm.at[pl.ds(0, CHUNK)], ar[b], si[b]).wait()
        pltpu.make_async_copy(col_hbm.at[pl.ds(0, CHUNK)], ac[b], si[b]).wait()
        pltpu.make_async_copy(bias_hbm.at[pl.ds(0, CHUNK)], bb[b], si[b]).wait()

    def _fetch_b(ci, b):
        off = ebase + ci * CHUNK
        pltpu.async_copy(row_hbm.at[pl.ds(off, CHUNK)], br[b], sb[b])

    def _wait_b(b):
        pltpu.make_async_copy(row_hbm.at[pl.ds(0, CHUNK)], br[b], sb[b]).wait()

    def _issue_gathers(b):
        pltpu.async_copy(q_hbm.at[ar[b]], qb[b], sg[b])
        pltpu.async_copy(kv_hbm.at[ac[b]], kvb[b], sg[b])

    def _wait_gathers(b):
        pltpu.make_async_copy(q_hbm.at[ar[b]], qb[b], sg[b]).wait()
        pltpu.make_async_copy(kv_hbm.at[ac[b]], kvb[b], sg[b]).wait()

    def _issue_scatter(b):
        pltpu.async_copy(s0, acc.at[br[b]], ss[b], add=True)

    def _wait_scatter(b):
        pltpu.make_async_copy(s0, acc.at[br[b]], ss[b]).wait()

    def _compute(b):
        bir, qr, kvr = bb[b], qb[b], kvb[b]

        @plsc.parallel_loop(0, CHUNK, unroll=4)
        def _edge(e):
            dot = qr[e, pl.ds(0, 16)] * kvr[e, pl.ds(0, 16)]
            for d in range(1, HEAD_DIM):
                dot = dot + qr[e, pl.ds(d * 16, 16)] * kvr[e, pl.ds(d * 16, 16)]
            p = jnp.exp(dot * SCALE + bir[e, :])
            for d in range(HEAD_DIM):
                s0[e, pl.ds(d * 16, 16)] = kvr[e, pl.ds(HIDDEN + d * 16, 16)] * p
            s0[e, pl.ds(HIDDEN, 16)] = p

    off0 = ebase
    pltpu.sync_copy(row_hbm.at[pl.ds(off0, CHUNK)], ar[0])
    pltpu.sync_copy(col_hbm.at[pl.ds(off0, CHUNK)], ac[0])
    pltpu.sync_copy(bias_hbm.at[pl.ds(off0, CHUNK)], bb[0])
    pltpu.sync_copy(row_hbm.at[pl.ds(off0, CHUNK)], br[0])
    _issue_gathers(0)
    _fetch_a(1, 1)

    def _pair(i, carry):
        for u in (0, 1):
            ci = 2 * i + u
            nu = 1 - u
            not_first = jnp.logical_or(i > 0, u > 0)
            not_last_pair = i < N_CHUNKS // 2 - 1
            _wait_gathers(u)

            if u == 0:
                _wait_a(1)
                _issue_gathers(1)
            else:
                @pl.when(not_last_pair)
                def _():
                    _wait_a(0)
                    _issue_gathers(0)

            @pl.when(not_first)
            def _():
                _wait_scatter(nu)
            if u == 0:
                _fetch_b(ci + 1, 1)
            else:
                @pl.when(not_last_pair)
                def _():
                    _fetch_b(ci + 1, 0)

            _compute(u)

            @pl.when(not_first)
            def _():
                _wait_b(u)
            _issue_scatter(u)

            @pl.when(not_last_pair)
            def _():
                _fetch_a(ci + 2, u)
        return carry

    lax.fori_loop(0, N_CHUNKS // 2, _pair, 0)
    _wait_scatter(1)

    plsc.subcore_barrier()
    out0 = cid * ACC_ROWS + r0
    for j in range(ROWS_PER_TILE // CHUNK):
        pltpu.sync_copy(acc.at[pl.ds(r0 + j * CHUNK, CHUNK)], s0)
        pltpu.sync_copy(s0, part_hbm.at[pl.ds(out0 + j * CHUNK, CHUNK)])


def _edge_pass(qt, kvt, bias, row, col):
    mesh = plsc.VectorSubcoreMesh(core_axis_name="c", subcore_axis_name="s")
    fn = pl.kernel(
        _edge_pass_body,
        out_type=[jax.ShapeDtypeStruct((NC * ACC_ROWS, PAYLOAD), jnp.float32)],
        mesh=mesh,
        scratch_types=(
            [pltpu.VMEM((CHUNK,), jnp.int32)] * 6
            + [pltpu.VMEM((CHUNK, HIDDEN), jnp.float32)] * 2
            + [pltpu.VMEM((CHUNK, 2 * HIDDEN), jnp.float32)] * 2
            + [pltpu.VMEM((CHUNK, HEADS), jnp.float32)] * 2
            + [pltpu.VMEM((CHUNK, PAYLOAD), jnp.float32)]
            + [pltpu.VMEM_SHARED((ACC_ROWS, PAYLOAD), jnp.float32)]
            + [pltpu.SemaphoreType.DMA] * 8
        ),
        compiler_params=pltpu.CompilerParams(use_tc_tiling_on_sc=False),
    )
    return fn(qt, kvt, bias, row, col)


def kernel(x, edge_index, edge_emb, Wq, bq, Wk, bk, Wv, bv, Wo, bo, Web, beb):
    c = jnp.arange(HIDDEN)
    perm = (c % HEADS) * HEAD_DIM + c // HEADS
    wqt = Wq.T[:, perm]
    wkvt = jnp.concatenate([Wk.T[:, perm], Wv.T[:, perm]], axis=1)
    wot = Wo.T[perm, :]
    bq_p = bq[perm]
    bkv_p = jnp.concatenate([bk[perm], bv[perm]])

    row = edge_index[0].astype(jnp.int32)
    col = edge_index[1].astype(jnp.int32)

    qt, kvt = _qkv_project(x, wqt, wkvt, bq_p, bkv_p)
    bias = _edge_bias(edge_emb, Web.T, beb)
    part, = _edge_pass(qt, kvt, bias, row, col)
    return _finalize(part, wot, bo)
